# baseline (speedup 1.0000x reference)
import functools

import jax
import jax.numpy as jnp
from jax import lax
from jax.experimental import pallas as pl
from jax.experimental.pallas import tpu as pltpu
from jax.experimental.pallas import tpu_sc as plsc

_N = 32768
_D = 768
_H = 64
_T = 64
_O = 64
_BLK = 2048


def _bias_gather_sc(headb, ti):
    info = plsc.get_sparse_core_info()
    nw = info.num_cores * info.num_subcores
    b_per_w = _N // nw
    mesh = plsc.VectorSubcoreMesh(core_axis_name="c", subcore_axis_name="s")

    @functools.partial(
        pl.kernel, mesh=mesh,
        out_type=jax.ShapeDtypeStruct((_N, _O), jnp.float32),
        scratch_types=[
            pltpu.VMEM((b_per_w,), jnp.int32),
            pltpu.VMEM((b_per_w, _O), jnp.float32),
            pltpu.SemaphoreType.DMA,
        ],
        compiler_params=pltpu.CompilerParams(use_tc_tiling_on_sc=False),
    )
    def k(table_hbm, idx_hbm, out_hbm, idx_v, rows_v, sem):
        wid = lax.axis_index("s") * info.num_cores + lax.axis_index("c")
        base = wid * b_per_w
        pltpu.sync_copy(idx_hbm.at[pl.ds(base, b_per_w)], idx_v)
        pltpu.async_copy(table_hbm.at[idx_v], rows_v, sem).wait()
        pltpu.sync_copy(rows_v, out_hbm.at[pl.ds(base, b_per_w)])

    return k(headb, ti)


def _trunk_head_kernel(x_ref, ti_ref, W1_ref, b1_ref, W2_ref, b2_ref,
                       W3_ref, b3_ref, W2d_ref, bt_ref, out_ref):
    x = x_ref[...].astype(jnp.bfloat16)
    h = jnp.tanh(jnp.dot(x, W1_ref[...], preferred_element_type=jnp.float32)
                 + b1_ref[...]).astype(jnp.bfloat16)
    h = jnp.tanh(jnp.dot(h, W2_ref[...], preferred_element_type=jnp.float32)
                 + b2_ref[...]).astype(jnp.bfloat16)
    h = jnp.tanh(jnp.dot(h, W3_ref[...], preferred_element_type=jnp.float32)
                 + b3_ref[...]).astype(jnp.bfloat16)
    ti = ti_ref[0, 0, :].reshape(_BLK, 1)
    grp = jax.lax.broadcasted_iota(jnp.int32, (_BLK, 8 * _H), 1) // _H
    h8 = jnp.concatenate([h] * 8, axis=1)
    hb8 = jnp.where(grp == ti % 8, h8, jnp.bfloat16(0))
    z = jnp.dot(hb8, W2d_ref[...], preferred_element_type=jnp.float32)
    z = jnp.where(grp == ti // 8, z, 0.0)
    w = (8 * _O) // 2
    while w >= _O:
        z = z[:, :w] + z[:, w:]
        w //= 2
    out_ref[...] = z + bt_ref[...]


def kernel(inputs, task_indices, W1, b1, W2, b2, W3, b3, headW, headb):
    n_blocks = _N // _BLK
    ti32 = task_indices.astype(jnp.int32)
    ti3 = ti32.reshape(n_blocks, 1, _BLK)
    bt = _bias_gather_sc(headb, ti32)
    W1 = W1.astype(jnp.bfloat16)
    W2 = W2.astype(jnp.bfloat16)
    W3 = W3.astype(jnp.bfloat16)
    W2d = (headW.reshape(8, 8, _H, _O).transpose(1, 2, 0, 3)
           .reshape(8 * _H, 8 * _O).astype(jnp.bfloat16))
    b1r = b1.reshape(1, _H)
    b2r = b2.reshape(1, _H)
    b3r = b3.reshape(1, _H)

    grid = (n_blocks,)
    out = pl.pallas_call(
        _trunk_head_kernel,
        grid=grid,
        in_specs=[
            pl.BlockSpec((_BLK, _D), lambda i: (i, 0)),
            pl.BlockSpec((1, 1, _BLK), lambda i: (i, 0, 0)),
            pl.BlockSpec((_D, _H), lambda i: (0, 0)),
            pl.BlockSpec((1, _H), lambda i: (0, 0)),
            pl.BlockSpec((_H, _H), lambda i: (0, 0)),
            pl.BlockSpec((1, _H), lambda i: (0, 0)),
            pl.BlockSpec((_H, _H), lambda i: (0, 0)),
            pl.BlockSpec((1, _H), lambda i: (0, 0)),
            pl.BlockSpec((8 * _H, 8 * _O), lambda i: (0, 0)),
            pl.BlockSpec((_BLK, _O), lambda i: (i, 0)),
        ],
        out_specs=pl.BlockSpec((_BLK, _O), lambda i: (i, 0)),
        out_shape=jax.ShapeDtypeStruct((_N, _O), jnp.float32),
        compiler_params=pltpu.CompilerParams(
            dimension_semantics=("parallel",)),
    )(inputs, ti3, W1, b1r, W2, b2r, W3, b3r, W2d, bt)
    return out


# speedup vs baseline: 1.4572x; 1.4572x over previous
import jax
import jax.numpy as jnp
from jax.experimental import pallas as pl
from jax.experimental.pallas import tpu as pltpu

_N = 32768
_D = 768
_H = 64
_T = 64
_O = 64
_BLK = 2048
_NBLK = _N // _BLK


def _trunk_head_kernel(x_hbm, ti_ref, W1_ref, b1_ref, W2_ref, b2_ref,
                       W3_ref, b3_ref, W2d_ref, hb_ref, out_ref,
                       xbuf, sem):
    i = pl.program_id(0)

    @pl.when(i == 0)
    def _prime():
        pltpu.make_async_copy(
            x_hbm.at[pl.ds(0, _BLK), :], xbuf.at[0], sem.at[0]).start()

    @pl.when(i + 1 < _NBLK)
    def _prefetch():
        slot_n = (i + 1) % 2
        pltpu.make_async_copy(
            x_hbm.at[pl.ds((i + 1) * _BLK, _BLK), :], xbuf.at[slot_n],
            sem.at[slot_n]).start()

    slot = i % 2
    pltpu.make_async_copy(
        x_hbm.at[pl.ds(i * _BLK, _BLK), :], xbuf.at[slot],
        sem.at[slot]).wait()

    x = xbuf[slot].astype(jnp.bfloat16)
    h = jnp.tanh(jnp.dot(x, W1_ref[...], preferred_element_type=jnp.float32)
                 + b1_ref[...]).astype(jnp.bfloat16)
    h = jnp.tanh(jnp.dot(h, W2_ref[...], preferred_element_type=jnp.float32)
                 + b2_ref[...]).astype(jnp.bfloat16)
    h = jnp.tanh(jnp.dot(h, W3_ref[...], preferred_element_type=jnp.float32)
                 + b3_ref[...]).astype(jnp.bfloat16)
    ti = ti_ref[0, 0, :].reshape(_BLK, 1)
    grp = jax.lax.broadcasted_iota(jnp.int32, (_BLK, 8 * _H), 1) // _H
    h8 = jnp.concatenate([h] * 8, axis=1)
    hb8 = jnp.where(grp == ti % 8, h8, jnp.bfloat16(0))
    z = jnp.dot(hb8, W2d_ref[...], preferred_element_type=jnp.float32)
    z = jnp.where(grp == ti // 8, z, 0.0)
    w = (8 * _O) // 2
    while w >= _O:
        z = z[:, :w] + z[:, w:]
        w //= 2
    onehot = (jax.lax.broadcasted_iota(jnp.int32, (_BLK, _T), 1)
              == ti).astype(jnp.float32)
    out_ref[...] = z + jnp.dot(onehot, hb_ref[...],
                               preferred_element_type=jnp.float32)


def kernel(inputs, task_indices, W1, b1, W2, b2, W3, b3, headW, headb):
    ti3 = task_indices.astype(jnp.int32).reshape(_NBLK, 1, _BLK)
    W1 = W1.astype(jnp.bfloat16)
    W2 = W2.astype(jnp.bfloat16)
    W3 = W3.astype(jnp.bfloat16)
    W2d = (headW.reshape(8, 8, _H, _O).transpose(1, 2, 0, 3)
           .reshape(8 * _H, 8 * _O).astype(jnp.bfloat16))
    b1r = b1.reshape(1, _H)
    b2r = b2.reshape(1, _H)
    b3r = b3.reshape(1, _H)

    out = pl.pallas_call(
        _trunk_head_kernel,
        grid=(_NBLK,),
        in_specs=[
            pl.BlockSpec(memory_space=pl.ANY),
            pl.BlockSpec((1, 1, _BLK), lambda i: (i, 0, 0)),
            pl.BlockSpec((_D, _H), lambda i: (0, 0)),
            pl.BlockSpec((1, _H), lambda i: (0, 0)),
            pl.BlockSpec((_H, _H), lambda i: (0, 0)),
            pl.BlockSpec((1, _H), lambda i: (0, 0)),
            pl.BlockSpec((_H, _H), lambda i: (0, 0)),
            pl.BlockSpec((1, _H), lambda i: (0, 0)),
            pl.BlockSpec((8 * _H, 8 * _O), lambda i: (0, 0)),
            pl.BlockSpec((_T, _O), lambda i: (0, 0)),
        ],
        out_specs=pl.BlockSpec((_BLK, _O), lambda i: (i, 0)),
        out_shape=jax.ShapeDtypeStruct((_N, _O), jnp.float32),
        scratch_shapes=[
            pltpu.VMEM((2, _BLK, _D), jnp.float32),
            pltpu.SemaphoreType.DMA((2,)),
        ],
        compiler_params=pltpu.CompilerParams(
            dimension_semantics=("arbitrary",)),
    )(inputs, ti3, W1, b1r, W2, b2r, W3, b3r, W2d, headb)
    return out
